# Initial kernel scaffold; baseline (speedup 1.0000x reference)
import jax
import jax.numpy as jnp
from jax.experimental import pallas as pl


def kernel(x, lora_indices, weight, lora_a_stacked_2d, lora_b_stacked):
    raise NotImplementedError("write your pallas kernel here")


# speedup vs baseline: 1.3549x; 1.3549x over previous
import functools

import jax
import jax.numpy as jnp
from jax import lax
from jax.experimental import pallas as pl
from jax.experimental.pallas import tpu as pltpu
from jax.experimental.pallas import tpu_sc as plsc

ORG_VOCAB = 100000
EXTRA_VOCAB = 256
FULL_VOCAB = ORG_VOCAB + EXTRA_VOCAB
D = 1024
R = 16
NLORA = 8
T = 16384

NC = 2
NS = 16
NW = NC * NS
TPW = T // NW
CW = 32
NCH = TPW // CW
ACH = 128
NACH = TPW // ACH

_sc_mesh = plsc.VectorSubcoreMesh(core_axis_name="c", subcore_axis_name="s")


@functools.partial(
    pl.kernel,
    mesh=_sc_mesh,
    compiler_params=pltpu.CompilerParams(use_tc_tiling_on_sc=False),
    out_type=[
        jax.ShapeDtypeStruct((T, D), jnp.float32),
        jax.ShapeDtypeStruct((T, R), jnp.float32),
    ],
    scratch_types=[
        pltpu.VMEM((TPW,), jnp.int32),
        pltpu.VMEM((TPW,), jnp.int32),
        pltpu.VMEM((NCH, CW), jnp.int32),
        pltpu.VMEM((NACH, ACH), jnp.int32),
        pltpu.VMEM((2, CW, D), jnp.float32),
        pltpu.VMEM((NACH, ACH, R), jnp.float32),
        pltpu.SemaphoreType.DMA,
        pltpu.SemaphoreType.DMA,
        pltpu.SemaphoreType.DMA,
    ],
)
def _sc_gather(x_hbm, li_hbm, w_hbm, la_hbm, base_hbm, a_hbm,
               x_v, li_v, widx, aidx, rowbuf, abuf, gsem, ssem, asem):
    wid = lax.axis_index("s") * NC + lax.axis_index("c")
    tok0 = wid * TPW

    pltpu.sync_copy(x_hbm.at[pl.ds(tok0, TPW)], x_v)
    pltpu.sync_copy(li_hbm.at[pl.ds(tok0, TPW)], li_v)

    for i in range(TPW // 16):
        xv = x_v[pl.ds(i * 16, 16)]
        lv = li_v[pl.ds(i * 16, 16)]
        widx[i // 2, pl.ds((i % 2) * 16, 16)] = xv
        aidx[i // 8, pl.ds((i % 8) * 16, 16)] = xv + lv * FULL_VOCAB

    a_copies = [
        pltpu.async_copy(la_hbm.at[aidx.at[j]], abuf.at[j], asem)
        for j in range(NACH)
    ]
    for j in range(NACH):
        a_copies[j].wait()
        pltpu.sync_copy(abuf.at[j], a_hbm.at[pl.ds(tok0 + j * ACH, ACH)])

    gathers = {}
    writes = {}
    gathers[0] = pltpu.async_copy(w_hbm.at[widx.at[0]], rowbuf.at[0], gsem)
    for j in range(NCH):
        if j + 1 < NCH:
            if j - 1 >= 0:
                writes[j - 1].wait()
            gathers[j + 1] = pltpu.async_copy(
                w_hbm.at[widx.at[j + 1]], rowbuf.at[(j + 1) % 2], gsem)
        gathers[j].wait()
        writes[j] = pltpu.async_copy(
            rowbuf.at[j % 2], base_hbm.at[pl.ds(tok0 + j * CW, CW)], ssem)
    writes[NCH - 2].wait()
    writes[NCH - 1].wait()


TB = 1024


def _combine_body(base_ref, a_ref, li_ref, bt_ref, o_ref):
    a8 = jnp.concatenate([a_ref[...]] * NLORA, axis=1)
    grp = lax.broadcasted_iota(jnp.int32, (TB, NLORA * R), 1) // R
    am = jnp.where(grp == li_ref[...], a8, 0.0)
    o_ref[...] = base_ref[...] + jnp.dot(
        am, bt_ref[...], preferred_element_type=jnp.float32)


def kernel(x, lora_indices, weight, lora_a_stacked_2d, lora_b_stacked):
    base, a = _sc_gather(x, lora_indices, weight, lora_a_stacked_2d)
    bt = lora_b_stacked[:, 0].transpose(0, 2, 1).reshape(NLORA * R, D)
    out = pl.pallas_call(
        _combine_body,
        grid=(T // TB,),
        in_specs=[
            pl.BlockSpec((TB, D), lambda i: (i, 0)),
            pl.BlockSpec((TB, R), lambda i: (i, 0)),
            pl.BlockSpec((TB, 1), lambda i: (i, 0)),
            pl.BlockSpec((NLORA * R, D), lambda i: (0, 0)),
        ],
        out_specs=pl.BlockSpec((TB, D), lambda i: (i, 0)),
        out_shape=jax.ShapeDtypeStruct((T, D), jnp.float32),
    )(base, a, lora_indices.reshape(T, 1), bt)
    return out


# speedup vs baseline: 2.1287x; 1.5711x over previous
import functools

import jax
import jax.numpy as jnp
from jax import lax
from jax.experimental import pallas as pl
from jax.experimental.pallas import tpu as pltpu
from jax.experimental.pallas import tpu_sc as plsc

ORG_VOCAB = 100000
EXTRA_VOCAB = 256
FULL_VOCAB = ORG_VOCAB + EXTRA_VOCAB
D = 1024
R = 16
NLORA = 8
T = 16384

NC = 2
NS = 16
NW = NC * NS
TPW = T // NW
CW = 32
NCH = TPW // CW
ACH = 128
NACH = TPW // ACH

_sc_mesh = plsc.VectorSubcoreMesh(core_axis_name="c", subcore_axis_name="s")


@functools.partial(
    pl.kernel,
    mesh=_sc_mesh,
    out_type=jax.ShapeDtypeStruct((T, D), jnp.float32),
    scratch_types=[
        pltpu.VMEM((TPW,), jnp.int32),
        pltpu.VMEM((NCH, CW), jnp.int32),
        pltpu.VMEM((2, CW, D), jnp.float32),
        pltpu.SemaphoreType.DMA,
        pltpu.SemaphoreType.DMA,
    ],
)
def _sc_gather_base(x_hbm, w_hbm, base_hbm, x_v, widx, rowbuf, gsem, ssem):
    wid = lax.axis_index("s") * NC + lax.axis_index("c")
    tok0 = wid * TPW

    pltpu.sync_copy(x_hbm.at[pl.ds(tok0, TPW)], x_v)

    for i in range(TPW // 16):
        xv = x_v[pl.ds(i * 16, 16)]
        widx[i // 2, pl.ds((i % 2) * 16, 16)] = xv

    gathers = {}
    writes = {}
    gathers[0] = pltpu.async_copy(w_hbm.at[widx.at[0]], rowbuf.at[0], gsem)
    for j in range(NCH):
        if j + 1 < NCH:
            if j - 1 >= 0:
                writes[j - 1].wait()
            gathers[j + 1] = pltpu.async_copy(
                w_hbm.at[widx.at[j + 1]], rowbuf.at[(j + 1) % 2], gsem)
        gathers[j].wait()
        writes[j] = pltpu.async_copy(
            rowbuf.at[j % 2], base_hbm.at[pl.ds(tok0 + j * CW, CW)], ssem)
    writes[NCH - 2].wait()
    writes[NCH - 1].wait()


@functools.partial(
    pl.kernel,
    mesh=_sc_mesh,
    compiler_params=pltpu.CompilerParams(use_tc_tiling_on_sc=False),
    out_type=jax.ShapeDtypeStruct((T, R), jnp.float32),
    scratch_types=[
        pltpu.VMEM((TPW,), jnp.int32),
        pltpu.VMEM((TPW,), jnp.int32),
        pltpu.VMEM((NACH, ACH), jnp.int32),
        pltpu.VMEM((NACH, ACH, R), jnp.float32),
        pltpu.SemaphoreType.DMA,
    ],
)
def _sc_gather_a(x_hbm, li_hbm, la_hbm, a_hbm, x_v, li_v, aidx, abuf, asem):
    wid = lax.axis_index("s") * NC + lax.axis_index("c")
    tok0 = wid * TPW

    pltpu.sync_copy(x_hbm.at[pl.ds(tok0, TPW)], x_v)
    pltpu.sync_copy(li_hbm.at[pl.ds(tok0, TPW)], li_v)

    for i in range(TPW // 16):
        xv = x_v[pl.ds(i * 16, 16)]
        lv = li_v[pl.ds(i * 16, 16)]
        aidx[i // 8, pl.ds((i % 8) * 16, 16)] = xv + lv * FULL_VOCAB

    a_copies = [
        pltpu.async_copy(la_hbm.at[aidx.at[j]], abuf.at[j], asem)
        for j in range(NACH)
    ]
    for j in range(NACH):
        a_copies[j].wait()
        pltpu.sync_copy(abuf.at[j], a_hbm.at[pl.ds(tok0 + j * ACH, ACH)])


TB = 1024


def _combine_body(base_ref, a_ref, li_ref, bt_ref, o_ref):
    a8 = jnp.concatenate([a_ref[...]] * NLORA, axis=1)
    grp = lax.broadcasted_iota(jnp.int32, (TB, NLORA * R), 1) // R
    am = jnp.where(grp == li_ref[...], a8, 0.0)
    o_ref[...] = base_ref[...] + jnp.dot(
        am, bt_ref[...], preferred_element_type=jnp.float32)


def kernel(x, lora_indices, weight, lora_a_stacked_2d, lora_b_stacked):
    base = _sc_gather_base(x, weight)
    a = _sc_gather_a(x, lora_indices, lora_a_stacked_2d)
    bt = lora_b_stacked[:, 0].transpose(0, 2, 1).reshape(NLORA * R, D)
    out = pl.pallas_call(
        _combine_body,
        grid=(T // TB,),
        in_specs=[
            pl.BlockSpec((TB, D), lambda i: (i, 0)),
            pl.BlockSpec((TB, R), lambda i: (i, 0)),
            pl.BlockSpec((TB, 1), lambda i: (i, 0)),
            pl.BlockSpec((NLORA * R, D), lambda i: (0, 0)),
        ],
        out_specs=pl.BlockSpec((TB, D), lambda i: (i, 0)),
        out_shape=jax.ShapeDtypeStruct((T, D), jnp.float32),
    )(base, a, lora_indices.reshape(T, 1), bt)
    return out


# speedup vs baseline: 7.1852x; 3.3753x over previous
import functools

import jax
import jax.numpy as jnp
from jax import lax
from jax.experimental import pallas as pl
from jax.experimental.pallas import tpu as pltpu
from jax.experimental.pallas import tpu_sc as plsc

ORG_VOCAB = 100000
EXTRA_VOCAB = 256
FULL_VOCAB = ORG_VOCAB + EXTRA_VOCAB
D = 1024
R = 16
NLORA = 8
T = 16384

NC = 2
NS = 16
NW = NC * NS
TPW = T // NW
CW = 32
NCH = TPW // CW
ACH = 128
NACH = TPW // ACH

NCB = FULL_VOCAB * NLORA // 128
HALF_STRIDE = NCB * 1024

_sc_mesh = plsc.VectorSubcoreMesh(core_axis_name="c", subcore_axis_name="s")


@functools.partial(
    pl.kernel,
    mesh=_sc_mesh,
    out_type=jax.ShapeDtypeStruct((T, D), jnp.float32),
    scratch_types=[
        pltpu.VMEM((TPW,), jnp.int32),
        pltpu.VMEM((NCH, CW), jnp.int32),
        pltpu.VMEM((2, CW, D), jnp.float32),
        pltpu.SemaphoreType.DMA,
        pltpu.SemaphoreType.DMA,
    ],
)
def _sc_gather_base(x_hbm, w_hbm, base_hbm, x_v, widx, rowbuf, gsem, ssem):
    wid = lax.axis_index("s") * NC + lax.axis_index("c")
    tok0 = wid * TPW

    pltpu.sync_copy(x_hbm.at[pl.ds(tok0, TPW)], x_v)

    for i in range(TPW // 16):
        xv = x_v[pl.ds(i * 16, 16)]
        widx[i // 2, pl.ds((i % 2) * 16, 16)] = xv

    gathers = {}
    writes = {}
    gathers[0] = pltpu.async_copy(w_hbm.at[widx.at[0]], rowbuf.at[0], gsem)
    for j in range(NCH):
        if j + 1 < NCH:
            if j - 1 >= 0:
                writes[j - 1].wait()
            gathers[j + 1] = pltpu.async_copy(
                w_hbm.at[widx.at[j + 1]], rowbuf.at[(j + 1) % 2], gsem)
        gathers[j].wait()
        writes[j] = pltpu.async_copy(
            rowbuf.at[j % 2], base_hbm.at[pl.ds(tok0 + j * CW, CW)], ssem)
    writes[NCH - 2].wait()
    writes[NCH - 1].wait()


@functools.partial(
    pl.kernel,
    mesh=_sc_mesh,
    out_type=jax.ShapeDtypeStruct((R, T), jnp.float32),
    scratch_types=[
        pltpu.VMEM((TPW,), jnp.int32),
        pltpu.VMEM((TPW,), jnp.int32),
        pltpu.VMEM((R * NACH, ACH), jnp.int32),
        pltpu.VMEM((R, TPW), jnp.float32),
        pltpu.SemaphoreType.DMA,
    ],
)
def _sc_gather_a(x_hbm, li_hbm, laf_hbm, at_hbm, x_v, li_v, aidx, abuf, asem):
    wid = lax.axis_index("s") * NC + lax.axis_index("c")
    tok0 = wid * TPW

    pltpu.sync_copy(x_hbm.at[pl.ds(tok0, TPW)], x_v)
    pltpu.sync_copy(li_hbm.at[pl.ds(tok0, TPW)], li_v)

    for i in range(TPW // 16):
        xv = x_v[pl.ds(i * 16, 16)]
        lv = li_v[pl.ds(i * 16, 16)]
        cv = xv + lv * FULL_VOCAB
        bv = ((cv >> 7) << 10) + (cv & 127)
        for r in range(R):
            off = (r // 8) * HALF_STRIDE + (r % 8) * 128
            aidx[r * NACH + i // 8, pl.ds((i % 8) * 16, 16)] = bv + off

    copies = []
    for r in range(R):
        for j in range(NACH):
            copies.append(pltpu.async_copy(
                laf_hbm.at[aidx.at[r * NACH + j]],
                abuf.at[r, pl.ds(j * ACH, ACH)],
                asem))
    for c in copies:
        c.wait()
    pltpu.sync_copy(abuf, at_hbm.at[:, pl.ds(tok0, TPW)])


TB = 1024


def _combine_body(base_ref, at_ref, li_ref, bt_ref, o_ref):
    a8 = jnp.concatenate([at_ref[...]] * NLORA, axis=0)
    grp = lax.broadcasted_iota(jnp.int32, (NLORA * R, TB), 0) // R
    am = jnp.where(grp == li_ref[...], a8, 0.0)
    o_ref[...] = base_ref[...] + lax.dot_general(
        am, bt_ref[...], (((0,), (0,)), ((), ())),
        preferred_element_type=jnp.float32)


def kernel(x, lora_indices, weight, lora_a_stacked_2d, lora_b_stacked):
    base = _sc_gather_base(x, weight)
    laf = (lora_a_stacked_2d.T
           .reshape(2, 8, NCB, 128)
           .transpose(0, 2, 1, 3)
           .reshape(2 * NCB * 8 * 128))
    at = _sc_gather_a(x, lora_indices, laf)
    bt = lora_b_stacked[:, 0].transpose(0, 2, 1).reshape(NLORA * R, D)
    out = pl.pallas_call(
        _combine_body,
        grid=(T // TB,),
        in_specs=[
            pl.BlockSpec((TB, D), lambda i: (i, 0)),
            pl.BlockSpec((R, TB), lambda i: (0, i)),
            pl.BlockSpec((1, TB), lambda i: (0, i)),
            pl.BlockSpec((NLORA * R, D), lambda i: (0, 0)),
        ],
        out_specs=pl.BlockSpec((TB, D), lambda i: (i, 0)),
        out_shape=jax.ShapeDtypeStruct((T, D), jnp.float32),
    )(base, at, lora_indices.reshape(1, T), bt)
    return out


# speedup vs baseline: 7.4917x; 1.0427x over previous
import functools

import jax
import jax.numpy as jnp
from jax import lax
from jax.experimental import pallas as pl
from jax.experimental.pallas import tpu as pltpu
from jax.experimental.pallas import tpu_sc as plsc

ORG_VOCAB = 100000
EXTRA_VOCAB = 256
FULL_VOCAB = ORG_VOCAB + EXTRA_VOCAB
D = 1024
R = 16
NLORA = 8
T = 16384

NC = 2
NS = 16
NW = NC * NS
TPW = T // NW
CW = 32
NCH = TPW // CW
ACH = 128
NACH = TPW // ACH

NCB = FULL_VOCAB * NLORA // 128
HALF_STRIDE = NCB * 1024

_sc_mesh = plsc.VectorSubcoreMesh(core_axis_name="c", subcore_axis_name="s")


@functools.partial(
    pl.kernel,
    mesh=_sc_mesh,
    out_type=[
        jax.ShapeDtypeStruct((T, D), jnp.float32),
        jax.ShapeDtypeStruct((R, T), jnp.float32),
    ],
    scratch_types=[
        pltpu.VMEM((TPW,), jnp.int32),
        pltpu.VMEM((TPW,), jnp.int32),
        pltpu.VMEM((NCH, CW), jnp.int32),
        pltpu.VMEM((R * NACH, ACH), jnp.int32),
        pltpu.VMEM((2, CW, D), jnp.float32),
        pltpu.VMEM((R, TPW), jnp.float32),
        pltpu.SemaphoreType.DMA,
        pltpu.SemaphoreType.DMA,
        pltpu.SemaphoreType.DMA,
    ],
)
def _sc_gather(x_hbm, li_hbm, w_hbm, laf_hbm, base_hbm, at_hbm,
               x_v, li_v, widx, aidx, rowbuf, abuf, gsem, ssem, asem):
    wid = lax.axis_index("s") * NC + lax.axis_index("c")
    tok0 = wid * TPW

    pltpu.sync_copy(x_hbm.at[pl.ds(tok0, TPW)], x_v)
    pltpu.sync_copy(li_hbm.at[pl.ds(tok0, TPW)], li_v)

    for i in range(TPW // 16):
        xv = x_v[pl.ds(i * 16, 16)]
        lv = li_v[pl.ds(i * 16, 16)]
        widx[i // 2, pl.ds((i % 2) * 16, 16)] = xv
        cv = xv + lv * FULL_VOCAB
        bv = ((cv >> 7) << 10) + (cv & 127)
        for r in range(R):
            off = (r // 8) * HALF_STRIDE + (r % 8) * 128
            aidx[r * NACH + i // 8, pl.ds((i % 8) * 16, 16)] = bv + off

    a_copies = []
    for r in range(R):
        for j in range(NACH):
            a_copies.append(pltpu.async_copy(
                laf_hbm.at[aidx.at[r * NACH + j]],
                abuf.at[r, pl.ds(j * ACH, ACH)],
                asem))

    gathers = {}
    writes = {}
    gathers[0] = pltpu.async_copy(w_hbm.at[widx.at[0]], rowbuf.at[0], gsem)
    for j in range(NCH):
        if j + 1 < NCH:
            if j - 1 >= 0:
                writes[j - 1].wait()
            gathers[j + 1] = pltpu.async_copy(
                w_hbm.at[widx.at[j + 1]], rowbuf.at[(j + 1) % 2], gsem)
        gathers[j].wait()
        writes[j] = pltpu.async_copy(
            rowbuf.at[j % 2], base_hbm.at[pl.ds(tok0 + j * CW, CW)], ssem)

    for c in a_copies:
        c.wait()
    pltpu.sync_copy(abuf, at_hbm.at[:, pl.ds(tok0, TPW)])
    writes[NCH - 2].wait()
    writes[NCH - 1].wait()


TB = 1024


def _combine_body(base_ref, at_ref, li_ref, bt_ref, o_ref):
    a8 = jnp.concatenate([at_ref[...]] * NLORA, axis=0)
    grp = lax.broadcasted_iota(jnp.int32, (NLORA * R, TB), 0) // R
    am = jnp.where(grp == li_ref[...], a8, 0.0)
    o_ref[...] = base_ref[...] + lax.dot_general(
        am, bt_ref[...], (((0,), (0,)), ((), ())),
        preferred_element_type=jnp.float32)


def kernel(x, lora_indices, weight, lora_a_stacked_2d, lora_b_stacked):
    laf = (lora_a_stacked_2d.T
           .reshape(2, 8, NCB, 128)
           .transpose(0, 2, 1, 3)
           .reshape(2 * NCB * 8 * 128))
    base, at = _sc_gather(x, lora_indices, weight, laf)
    bt = lora_b_stacked[:, 0].transpose(0, 2, 1).reshape(NLORA * R, D)
    out = pl.pallas_call(
        _combine_body,
        grid=(T // TB,),
        in_specs=[
            pl.BlockSpec((TB, D), lambda i: (i, 0)),
            pl.BlockSpec((R, TB), lambda i: (0, i)),
            pl.BlockSpec((1, TB), lambda i: (0, i)),
            pl.BlockSpec((NLORA * R, D), lambda i: (0, 0)),
        ],
        out_specs=pl.BlockSpec((TB, D), lambda i: (i, 0)),
        out_shape=jax.ShapeDtypeStruct((T, D), jnp.float32),
    )(base, at, lora_indices.reshape(1, T), bt)
    return out


# speedup vs baseline: 7.4985x; 1.0009x over previous
import functools

import jax
import jax.numpy as jnp
from jax import lax
from jax.experimental import pallas as pl
from jax.experimental.pallas import tpu as pltpu
from jax.experimental.pallas import tpu_sc as plsc

ORG_VOCAB = 100000
EXTRA_VOCAB = 256
FULL_VOCAB = ORG_VOCAB + EXTRA_VOCAB
D = 1024
R = 16
NLORA = 8
T = 16384

NC = 2
NS = 16
NW = NC * NS
TPW = T // NW
CW = 32
NCH = TPW // CW
ACH = 128
NACH = TPW // ACH

NCB = FULL_VOCAB * NLORA // 128
HALF_STRIDE = NCB * 1024

_sc_mesh = plsc.VectorSubcoreMesh(core_axis_name="c", subcore_axis_name="s")


@functools.partial(
    pl.kernel,
    mesh=_sc_mesh,
    out_type=[
        jax.ShapeDtypeStruct((T, D), jnp.float32),
        jax.ShapeDtypeStruct((R, T), jnp.float32),
    ],
    scratch_types=[
        pltpu.VMEM((TPW,), jnp.int32),
        pltpu.VMEM((TPW,), jnp.int32),
        pltpu.VMEM((NCH, CW), jnp.int32),
        pltpu.VMEM((R * NACH, ACH), jnp.int32),
        pltpu.VMEM((2, CW, D), jnp.float32),
        pltpu.VMEM((R, TPW), jnp.float32),
        pltpu.SemaphoreType.DMA,
        pltpu.SemaphoreType.DMA,
        pltpu.SemaphoreType.DMA,
    ],
)
def _sc_gather(x_hbm, li_hbm, w_hbm, laf_hbm, base_hbm, at_hbm,
               x_v, li_v, widx, aidx, rowbuf, abuf, gsem, ssem, asem):
    wid = lax.axis_index("s") * NC + lax.axis_index("c")
    tok0 = wid * TPW

    pltpu.sync_copy(x_hbm.at[pl.ds(tok0, TPW)], x_v)
    pltpu.sync_copy(li_hbm.at[pl.ds(tok0, TPW)], li_v)

    for i in range(TPW // 16):
        xv = x_v[pl.ds(i * 16, 16)]
        lv = li_v[pl.ds(i * 16, 16)]
        widx[i // 2, pl.ds((i % 2) * 16, 16)] = xv
        cv = xv + lv * FULL_VOCAB
        bv = ((cv >> 7) << 10) + (cv & 127)
        for r in range(R):
            off = (r // 8) * HALF_STRIDE + (r % 8) * 128
            aidx[r * NACH + i // 8, pl.ds((i % 8) * 16, 16)] = bv + off

    a_copies = []
    for r in range(R):
        for j in range(NACH):
            a_copies.append(pltpu.async_copy(
                laf_hbm.at[aidx.at[r * NACH + j]],
                abuf.at[r, pl.ds(j * ACH, ACH)],
                asem))

    gathers = {}
    writes = {}
    gathers[0] = pltpu.async_copy(w_hbm.at[widx.at[0]], rowbuf.at[0], gsem)
    for j in range(NCH):
        if j + 1 < NCH:
            if j - 1 >= 0:
                writes[j - 1].wait()
            gathers[j + 1] = pltpu.async_copy(
                w_hbm.at[widx.at[j + 1]], rowbuf.at[(j + 1) % 2], gsem)
        gathers[j].wait()
        writes[j] = pltpu.async_copy(
            rowbuf.at[j % 2], base_hbm.at[pl.ds(tok0 + j * CW, CW)], ssem)

    for c in a_copies:
        c.wait()
    pltpu.sync_copy(abuf, at_hbm.at[:, pl.ds(tok0, TPW)])
    writes[NCH - 2].wait()
    writes[NCH - 1].wait()


TB = 1024


def _combine_body(base_ref, at_ref, li_ref, bt_ref, o_ref):
    a8 = jnp.concatenate([at_ref[...]] * NLORA, axis=0)
    grp = lax.broadcasted_iota(jnp.int32, (NLORA * R, TB), 0) // R
    am = jnp.where(grp == li_ref[...], a8, 0.0)
    o_ref[...] = base_ref[...] + lax.dot_general(
        am.astype(jnp.bfloat16), bt_ref[...].astype(jnp.bfloat16),
        (((0,), (0,)), ((), ())),
        preferred_element_type=jnp.float32)


def kernel(x, lora_indices, weight, lora_a_stacked_2d, lora_b_stacked):
    laf = (lora_a_stacked_2d.T
           .reshape(2, 8, NCB, 128)
           .transpose(0, 2, 1, 3)
           .reshape(2 * NCB * 8 * 128))
    base, at = _sc_gather(x, lora_indices, weight, laf)
    bt = lora_b_stacked[:, 0].transpose(0, 2, 1).reshape(NLORA * R, D)
    out = pl.pallas_call(
        _combine_body,
        grid=(T // TB,),
        in_specs=[
            pl.BlockSpec((TB, D), lambda i: (i, 0)),
            pl.BlockSpec((R, TB), lambda i: (0, i)),
            pl.BlockSpec((1, TB), lambda i: (0, i)),
            pl.BlockSpec((NLORA * R, D), lambda i: (0, 0)),
        ],
        out_specs=pl.BlockSpec((TB, D), lambda i: (i, 0)),
        out_shape=jax.ShapeDtypeStruct((T, D), jnp.float32),
    )(base, at, lora_indices.reshape(1, T), bt)
    return out
